# baseline (speedup 1.0000x reference)
import functools

import jax
import jax.numpy as jnp
from jax import lax
from jax.experimental import pallas as pl
from jax.experimental.pallas import tpu as pltpu
from jax.experimental.pallas import tpu_sc as plsc

_VOCAB = 100000
_D = 64
_B = 1024
_V_BLK = 2048
_NB = pl.cdiv(_VOCAB, _V_BLK)
_TAIL = _VOCAB - (_NB - 1) * _V_BLK
_TAIL_A = (_TAIL // 128) * 128
_TAIL_B = _TAIL - _TAIL_A
_NBUF = 4


@functools.lru_cache(maxsize=None)
def _make_sc_gather(D, B):
    info = plsc.get_sparse_core_info()
    NC, NS = info.num_cores, info.num_subcores
    NW = NC * NS
    assert B % (8 * NW) == 0
    b_per_w = B // NW
    mesh = plsc.VectorSubcoreMesh(core_axis_name="c", subcore_axis_name="s")

    @functools.partial(
        pl.kernel,
        mesh=mesh,
        out_type=jax.ShapeDtypeStruct((B, D), jnp.float32),
        scratch_types=[
            pltpu.VMEM((b_per_w,), jnp.int32),
            pltpu.VMEM((b_per_w, D), jnp.float32),
            pltpu.SemaphoreType.DMA,
        ],
        compiler_params=pltpu.CompilerParams(use_tc_tiling_on_sc=False),
    )
    def gather(table_hbm, idx_hbm, out_hbm, idx_v, rows_v, sem):
        wid = lax.axis_index("s") * NC + lax.axis_index("c")
        base = wid * b_per_w
        pltpu.sync_copy(idx_hbm.at[pl.ds(base, b_per_w)], idx_v)
        pltpu.async_copy(table_hbm.at[idx_v], rows_v, sem).wait()
        pltpu.sync_copy(rows_v, out_hbm.at[pl.ds(base, b_per_w)])

    return gather


def _mm_body(x_ref, w_ref, b_ref, o_hbm, scratch, tail_buf, sems, tail_sem):
    i = pl.program_id(0)
    slot = lax.rem(i, _NBUF)

    @pl.when(i >= _NBUF)
    def _():
        pltpu.make_async_copy(
            scratch.at[slot],
            o_hbm.at[:, pl.ds(0, _V_BLK)],
            sems.at[slot],
        ).wait()

    val = jnp.dot(
        x_ref[...], w_ref[...], preferred_element_type=jnp.float32
    ) + b_ref[...]
    scratch[slot] = val

    @pl.when(i < _NB - 1)
    def _():
        pltpu.make_async_copy(
            scratch.at[slot],
            o_hbm.at[:, pl.ds(i * _V_BLK, _V_BLK)],
            sems.at[slot],
        ).start()

    @pl.when(i == _NB - 1)
    def _():
        tail_buf[...] = val[:, _TAIL_A:_TAIL]
        pltpu.make_async_copy(
            scratch.at[slot, :, pl.ds(0, _TAIL_A)],
            o_hbm.at[:, pl.ds((_NB - 1) * _V_BLK, _TAIL_A)],
            sems.at[slot],
        ).start()
        pltpu.make_async_copy(
            tail_buf,
            o_hbm.at[:, pl.ds((_NB - 1) * _V_BLK + _TAIL_A, _TAIL_B)],
            tail_sem,
        ).start()
        for j in range(_NB - _NBUF, _NB):
            k = j % _NBUF
            if j == _NB - 1:
                src = scratch.at[k, :, pl.ds(0, _TAIL_A)]
                dst = o_hbm.at[:, pl.ds(0, _TAIL_A)]
            else:
                src = scratch.at[k]
                dst = o_hbm.at[:, pl.ds(0, _V_BLK)]
            pltpu.make_async_copy(src, dst, sems.at[k]).wait()
        pltpu.make_async_copy(
            tail_buf,
            o_hbm.at[:, pl.ds((_NB - 1) * _V_BLK + _TAIL_A, _TAIL_B)],
            tail_sem,
        ).wait()


def _tc_project(x, Wt, b2d):
    return pl.pallas_call(
        _mm_body,
        grid=(_NB,),
        in_specs=[
            pl.BlockSpec((_B, _D), lambda i: (0, 0)),
            pl.BlockSpec((_D, _V_BLK), lambda i: (0, i)),
            pl.BlockSpec((1, _V_BLK), lambda i: (0, i)),
        ],
        out_specs=pl.BlockSpec(memory_space=pltpu.MemorySpace.HBM),
        out_shape=jax.ShapeDtypeStruct((_B, _VOCAB), jnp.float32),
        scratch_shapes=[
            pltpu.VMEM((_NBUF, _B, _V_BLK), jnp.float32),
            pltpu.VMEM((_B, _TAIL_B), jnp.float32),
            pltpu.SemaphoreType.DMA((_NBUF,)),
            pltpu.SemaphoreType.DMA,
        ],
        compiler_params=pltpu.CompilerParams(
            dimension_semantics=("arbitrary",),
        ),
    )(x, Wt, b2d)


@jax.jit
def kernel(inputs, embeddings, W, b):
    idx = inputs.astype(jnp.int32)
    embeds = _make_sc_gather(_D, _B)(embeddings, idx)
    return _tc_project(embeds, W.T, b.reshape(1, _VOCAB))


# speedup vs baseline: 1.0010x; 1.0010x over previous
import functools

import jax
import jax.numpy as jnp
from jax import lax
from jax.experimental import pallas as pl
from jax.experimental.pallas import tpu as pltpu
from jax.experimental.pallas import tpu_sc as plsc

_VOCAB = 100000
_D = 64
_B = 1024
_V_BLK = 2048
_NB = pl.cdiv(_VOCAB, _V_BLK)
_TAIL = _VOCAB - (_NB - 1) * _V_BLK
_TAIL_A = (_TAIL // 128) * 128
_TAIL_B = _TAIL - _TAIL_A
_NBUF = 4


@functools.lru_cache(maxsize=None)
def _make_sc_gather(D, B):
    info = plsc.get_sparse_core_info()
    NC, NS = info.num_cores, info.num_subcores
    NW = NC * NS
    assert B % (8 * NW) == 0
    b_per_w = B // NW
    mesh = plsc.VectorSubcoreMesh(core_axis_name="c", subcore_axis_name="s")

    @functools.partial(
        pl.kernel,
        mesh=mesh,
        out_type=jax.ShapeDtypeStruct((B, D), jnp.float32),
        scratch_types=[
            pltpu.VMEM((b_per_w,), jnp.int32),
            pltpu.VMEM((b_per_w, D), jnp.float32),
            pltpu.SemaphoreType.DMA,
        ],
        compiler_params=pltpu.CompilerParams(use_tc_tiling_on_sc=False),
    )
    def gather(table_hbm, idx_hbm, out_hbm, idx_v, rows_v, sem):
        wid = lax.axis_index("s") * NC + lax.axis_index("c")
        base = wid * b_per_w
        pltpu.sync_copy(idx_hbm.at[pl.ds(base, b_per_w)], idx_v)
        pltpu.async_copy(table_hbm.at[idx_v], rows_v, sem).wait()
        pltpu.sync_copy(rows_v, out_hbm.at[pl.ds(base, b_per_w)])

    return gather


def _mm_body(x_ref, w_ref, b_ref, o_hbm, scratch, tail_buf, sems, tail_sem):
    i = pl.program_id(0)
    slot = lax.rem(i, _NBUF)

    @pl.when(i >= _NBUF)
    def _():
        pltpu.make_async_copy(
            scratch.at[slot],
            o_hbm.at[:, pl.ds(0, _V_BLK)],
            sems.at[slot],
        ).wait()

    val = jnp.broadcast_to(b_ref[...], (_B, _V_BLK))
    scratch[slot] = val

    @pl.when(i < _NB - 1)
    def _():
        pltpu.make_async_copy(
            scratch.at[slot],
            o_hbm.at[:, pl.ds(i * _V_BLK, _V_BLK)],
            sems.at[slot],
        ).start()

    @pl.when(i == _NB - 1)
    def _():
        tail_buf[...] = val[:, _TAIL_A:_TAIL]
        pltpu.make_async_copy(
            scratch.at[slot, :, pl.ds(0, _TAIL_A)],
            o_hbm.at[:, pl.ds((_NB - 1) * _V_BLK, _TAIL_A)],
            sems.at[slot],
        ).start()
        pltpu.make_async_copy(
            tail_buf,
            o_hbm.at[:, pl.ds((_NB - 1) * _V_BLK + _TAIL_A, _TAIL_B)],
            tail_sem,
        ).start()
        for j in range(_NB - _NBUF, _NB):
            k = j % _NBUF
            if j == _NB - 1:
                src = scratch.at[k, :, pl.ds(0, _TAIL_A)]
                dst = o_hbm.at[:, pl.ds(0, _TAIL_A)]
            else:
                src = scratch.at[k]
                dst = o_hbm.at[:, pl.ds(0, _V_BLK)]
            pltpu.make_async_copy(src, dst, sems.at[k]).wait()
        pltpu.make_async_copy(
            tail_buf,
            o_hbm.at[:, pl.ds((_NB - 1) * _V_BLK + _TAIL_A, _TAIL_B)],
            tail_sem,
        ).wait()


def _tc_project(x, Wt, b2d):
    return pl.pallas_call(
        _mm_body,
        grid=(_NB,),
        in_specs=[
            pl.BlockSpec((_B, _D), lambda i: (0, 0)),
            pl.BlockSpec((_D, _V_BLK), lambda i: (0, i)),
            pl.BlockSpec((1, _V_BLK), lambda i: (0, i)),
        ],
        out_specs=pl.BlockSpec(memory_space=pltpu.MemorySpace.HBM),
        out_shape=jax.ShapeDtypeStruct((_B, _VOCAB), jnp.float32),
        scratch_shapes=[
            pltpu.VMEM((_NBUF, _B, _V_BLK), jnp.float32),
            pltpu.VMEM((_B, _TAIL_B), jnp.float32),
            pltpu.SemaphoreType.DMA((_NBUF,)),
            pltpu.SemaphoreType.DMA,
        ],
        compiler_params=pltpu.CompilerParams(
            dimension_semantics=("arbitrary",),
        ),
    )(x, Wt, b2d)


@jax.jit
def kernel(inputs, embeddings, W, b):
    idx = inputs.astype(jnp.int32)
    embeds = _make_sc_gather(_D, _B)(embeddings, idx)
    return _tc_project(embeds, W.T, b.reshape(1, _VOCAB))


# speedup vs baseline: 1.0021x; 1.0011x over previous
import functools

import jax
import jax.numpy as jnp
from jax import lax
from jax.experimental import pallas as pl
from jax.experimental.pallas import tpu as pltpu
from jax.experimental.pallas import tpu_sc as plsc

_VOCAB = 100000
_D = 64
_B = 1024
_M_BLK = 16
_NB = _B // _M_BLK
_NBUF = 3


@functools.lru_cache(maxsize=None)
def _make_sc_gather(D, B):
    info = plsc.get_sparse_core_info()
    NC, NS = info.num_cores, info.num_subcores
    NW = NC * NS
    assert B % (8 * NW) == 0
    b_per_w = B // NW
    mesh = plsc.VectorSubcoreMesh(core_axis_name="c", subcore_axis_name="s")

    @functools.partial(
        pl.kernel,
        mesh=mesh,
        out_type=jax.ShapeDtypeStruct((B, D), jnp.float32),
        scratch_types=[
            pltpu.VMEM((b_per_w,), jnp.int32),
            pltpu.VMEM((b_per_w, D), jnp.float32),
            pltpu.SemaphoreType.DMA,
        ],
        compiler_params=pltpu.CompilerParams(use_tc_tiling_on_sc=False),
    )
    def gather(table_hbm, idx_hbm, out_hbm, idx_v, rows_v, sem):
        wid = lax.axis_index("s") * NC + lax.axis_index("c")
        base = wid * b_per_w
        pltpu.sync_copy(idx_hbm.at[pl.ds(base, b_per_w)], idx_v)
        pltpu.async_copy(table_hbm.at[idx_v], rows_v, sem).wait()
        pltpu.sync_copy(rows_v, out_hbm.at[pl.ds(base, b_per_w)])

    return gather


def _mm_body(x_ref, w_ref, b_ref, o_hbm, scratch, sems):
    i = pl.program_id(0)
    slot = lax.rem(i, _NBUF)

    @pl.when(i >= _NBUF)
    def _():
        pltpu.make_async_copy(
            scratch.at[slot],
            o_hbm.at[pl.ds(0, _M_BLK), :],
            sems.at[slot],
        ).wait()

    scratch[slot] = jnp.dot(
        x_ref[...], w_ref[...], preferred_element_type=jnp.float32
    ) + b_ref[...]

    pltpu.make_async_copy(
        scratch.at[slot],
        o_hbm.at[pl.ds(i * _M_BLK, _M_BLK), :],
        sems.at[slot],
    ).start()

    @pl.when(i == _NB - 1)
    def _():
        for k in range(_NBUF):
            pltpu.make_async_copy(
                scratch.at[k],
                o_hbm.at[pl.ds(0, _M_BLK), :],
                sems.at[k],
            ).wait()


def _tc_project(x, Wt, b2d):
    return pl.pallas_call(
        _mm_body,
        grid=(_NB,),
        in_specs=[
            pl.BlockSpec((_M_BLK, _D), lambda i: (i, 0)),
            pl.BlockSpec((_D, _VOCAB), lambda i: (0, 0)),
            pl.BlockSpec((1, _VOCAB), lambda i: (0, 0)),
        ],
        out_specs=pl.BlockSpec(memory_space=pltpu.MemorySpace.HBM),
        out_shape=jax.ShapeDtypeStruct((_B, _VOCAB), jnp.float32),
        scratch_shapes=[
            pltpu.VMEM((_NBUF, _M_BLK, _VOCAB), jnp.float32),
            pltpu.SemaphoreType.DMA((_NBUF,)),
        ],
        compiler_params=pltpu.CompilerParams(
            dimension_semantics=("arbitrary",),
            vmem_limit_bytes=100 * 1024 * 1024,
        ),
    )(x, Wt, b2d)


@jax.jit
def kernel(inputs, embeddings, W, b):
    idx = inputs.astype(jnp.int32)
    embeds = _make_sc_gather(_D, _B)(embeddings, idx)
    return _tc_project(embeds, W.T, b.reshape(1, _VOCAB))


# speedup vs baseline: 1.0061x; 1.0039x over previous
import functools

import jax
import jax.numpy as jnp
from jax import lax
from jax.experimental import pallas as pl
from jax.experimental.pallas import tpu as pltpu
from jax.experimental.pallas import tpu_sc as plsc

_VOCAB = 100000
_D = 64
_B = 1024
_M_BLK = 16
_NB = _B // _M_BLK
_NBUF = 3


@functools.lru_cache(maxsize=None)
def _make_sc_gather(D, B):
    info = plsc.get_sparse_core_info()
    NC, NS = info.num_cores, info.num_subcores
    NW = NC * NS
    assert B % (8 * NW) == 0
    b_per_w = B // NW
    mesh = plsc.VectorSubcoreMesh(core_axis_name="c", subcore_axis_name="s")

    @functools.partial(
        pl.kernel,
        mesh=mesh,
        out_type=jax.ShapeDtypeStruct((B, D), jnp.float32),
        scratch_types=[
            pltpu.VMEM((b_per_w,), jnp.int32),
            pltpu.VMEM((b_per_w, D), jnp.float32),
            pltpu.SemaphoreType.DMA,
        ],
        compiler_params=pltpu.CompilerParams(use_tc_tiling_on_sc=False),
    )
    def gather(table_hbm, idx_hbm, out_hbm, idx_v, rows_v, sem):
        wid = lax.axis_index("s") * NC + lax.axis_index("c")
        base = wid * b_per_w
        pltpu.sync_copy(idx_hbm.at[pl.ds(base, b_per_w)], idx_v)
        pltpu.async_copy(table_hbm.at[idx_v], rows_v, sem).wait()
        pltpu.sync_copy(rows_v, out_hbm.at[pl.ds(base, b_per_w)])

    return gather


def _mm_body(x_ref, w_ref, b_ref, o_hbm, scratch, sems):
    i = pl.program_id(0)
    slot = lax.rem(i, _NBUF)

    @pl.when(i >= _NBUF)
    def _():
        pltpu.make_async_copy(
            scratch.at[slot],
            o_hbm.at[pl.ds(0, _M_BLK), :],
            sems.at[slot],
        ).wait()


    pltpu.make_async_copy(
        scratch.at[slot],
        o_hbm.at[pl.ds(i * _M_BLK, _M_BLK), :],
        sems.at[slot],
    ).start()

    @pl.when(i == _NB - 1)
    def _():
        for k in range(_NBUF):
            pltpu.make_async_copy(
                scratch.at[k],
                o_hbm.at[pl.ds(0, _M_BLK), :],
                sems.at[k],
            ).wait()


def _tc_project(x, Wt, b2d):
    return pl.pallas_call(
        _mm_body,
        grid=(_NB,),
        in_specs=[
            pl.BlockSpec((_M_BLK, _D), lambda i: (i, 0)),
            pl.BlockSpec((_D, _VOCAB), lambda i: (0, 0)),
            pl.BlockSpec((1, _VOCAB), lambda i: (0, 0)),
        ],
        out_specs=pl.BlockSpec(memory_space=pltpu.MemorySpace.HBM),
        out_shape=jax.ShapeDtypeStruct((_B, _VOCAB), jnp.float32),
        scratch_shapes=[
            pltpu.VMEM((_NBUF, _M_BLK, _VOCAB), jnp.float32),
            pltpu.SemaphoreType.DMA((_NBUF,)),
        ],
        compiler_params=pltpu.CompilerParams(
            dimension_semantics=("arbitrary",),
            vmem_limit_bytes=100 * 1024 * 1024,
        ),
    )(x, Wt, b2d)


@jax.jit
def kernel(inputs, embeddings, W, b):
    idx = inputs.astype(jnp.int32)
    embeds = _make_sc_gather(_D, _B)(embeddings, idx)
    return _tc_project(embeds, W.T, b.reshape(1, _VOCAB))


# speedup vs baseline: 1.0072x; 1.0011x over previous
import functools

import jax
import jax.numpy as jnp
from jax import lax
from jax.experimental import pallas as pl
from jax.experimental.pallas import tpu as pltpu
from jax.experimental.pallas import tpu_sc as plsc

_VOCAB = 100000
_D = 64
_B = 1024
_M_BLK = 16
_NB = _B // _M_BLK
_NBUF = 3


@functools.lru_cache(maxsize=None)
def _make_sc_gather(D, B):
    info = plsc.get_sparse_core_info()
    NC, NS = info.num_cores, info.num_subcores
    NW = NC * NS
    assert B % (8 * NW) == 0
    b_per_w = B // NW
    mesh = plsc.VectorSubcoreMesh(core_axis_name="c", subcore_axis_name="s")

    @functools.partial(
        pl.kernel,
        mesh=mesh,
        out_type=jax.ShapeDtypeStruct((B, D), jnp.float32),
        scratch_types=[
            pltpu.VMEM((b_per_w,), jnp.int32),
            pltpu.VMEM((b_per_w, D), jnp.float32),
            pltpu.SemaphoreType.DMA,
        ],
        compiler_params=pltpu.CompilerParams(use_tc_tiling_on_sc=False),
    )
    def gather(table_hbm, idx_hbm, out_hbm, idx_v, rows_v, sem):
        wid = lax.axis_index("s") * NC + lax.axis_index("c")
        base = wid * b_per_w
        pltpu.sync_copy(idx_hbm.at[pl.ds(base, b_per_w)], idx_v)
        pltpu.async_copy(table_hbm.at[idx_v], rows_v, sem).wait()
        pltpu.sync_copy(rows_v, out_hbm.at[pl.ds(base, b_per_w)])

    return gather


_NSTRIPE = 4
_S_ROWS = _M_BLK // _NSTRIPE


def _mm_body(x_ref, w_ref, b_ref, o_hbm, scratch, sems):
    i = pl.program_id(0)
    slot = lax.rem(i, _NBUF)

    @pl.when(i >= _NBUF)
    def _():
        for s in range(_NSTRIPE):
            pltpu.make_async_copy(
                scratch.at[slot, pl.ds(s * _S_ROWS, _S_ROWS), :],
                o_hbm.at[pl.ds(s * _S_ROWS, _S_ROWS), :],
                sems.at[slot, s],
            ).wait()


    for s in range(_NSTRIPE):
        pltpu.make_async_copy(
            scratch.at[slot, pl.ds(s * _S_ROWS, _S_ROWS), :],
            o_hbm.at[pl.ds(i * _M_BLK + s * _S_ROWS, _S_ROWS), :],
            sems.at[slot, s],
        ).start()

    @pl.when(i == _NB - 1)
    def _():
        for k in range(_NBUF):
            for s in range(_NSTRIPE):
                pltpu.make_async_copy(
                    scratch.at[k, pl.ds(s * _S_ROWS, _S_ROWS), :],
                    o_hbm.at[pl.ds(s * _S_ROWS, _S_ROWS), :],
                    sems.at[k, s],
                ).wait()


def _tc_project(x, Wt, b2d):
    return pl.pallas_call(
        _mm_body,
        grid=(_NB,),
        in_specs=[
            pl.BlockSpec((_M_BLK, _D), lambda i: (i, 0)),
            pl.BlockSpec((_D, _VOCAB), lambda i: (0, 0)),
            pl.BlockSpec((1, _VOCAB), lambda i: (0, 0)),
        ],
        out_specs=pl.BlockSpec(memory_space=pltpu.MemorySpace.HBM),
        out_shape=jax.ShapeDtypeStruct((_B, _VOCAB), jnp.float32),
        scratch_shapes=[
            pltpu.VMEM((_NBUF, _M_BLK, _VOCAB), jnp.float32),
            pltpu.SemaphoreType.DMA((_NBUF, _NSTRIPE)),
        ],
        compiler_params=pltpu.CompilerParams(
            dimension_semantics=("arbitrary",),
            vmem_limit_bytes=100 * 1024 * 1024,
        ),
    )(x, Wt, b2d)


@jax.jit
def kernel(inputs, embeddings, W, b):
    idx = inputs.astype(jnp.int32)
    embeds = _make_sc_gather(_D, _B)(embeddings, idx)
    return _tc_project(embeds, W.T, b.reshape(1, _VOCAB))


# speedup vs baseline: 1.7709x; 1.7583x over previous
import functools

import jax
import jax.numpy as jnp
from jax import lax
from jax.experimental import pallas as pl
from jax.experimental.pallas import tpu as pltpu
from jax.experimental.pallas import tpu_sc as plsc

_VOCAB = 100000
_D = 64
_B = 1024
_V_BLK = 2048
_NB = pl.cdiv(_VOCAB, _V_BLK)


@functools.lru_cache(maxsize=None)
def _make_sc_gather(D, B):
    info = plsc.get_sparse_core_info()
    NC, NS = info.num_cores, info.num_subcores
    NW = NC * NS
    assert B % (8 * NW) == 0
    b_per_w = B // NW
    mesh = plsc.VectorSubcoreMesh(core_axis_name="c", subcore_axis_name="s")

    @functools.partial(
        pl.kernel,
        mesh=mesh,
        out_type=jax.ShapeDtypeStruct((B, D), jnp.float32),
        scratch_types=[
            pltpu.VMEM((b_per_w,), jnp.int32),
            pltpu.VMEM((b_per_w, D), jnp.float32),
            pltpu.SemaphoreType.DMA,
        ],
        compiler_params=pltpu.CompilerParams(use_tc_tiling_on_sc=False),
    )
    def gather(table_hbm, idx_hbm, out_hbm, idx_v, rows_v, sem):
        wid = lax.axis_index("s") * NC + lax.axis_index("c")
        base = wid * b_per_w
        pltpu.sync_copy(idx_hbm.at[pl.ds(base, b_per_w)], idx_v)
        pltpu.async_copy(table_hbm.at[idx_v], rows_v, sem).wait()
        pltpu.sync_copy(rows_v, out_hbm.at[pl.ds(base, b_per_w)])

    return gather


def _mm_body(xt_ref, w_ref, b_ref, o_ref):
    o_ref[...] = jnp.dot(
        w_ref[...], xt_ref[...], preferred_element_type=jnp.float32
    ) + b_ref[...]


def _tc_project_t(xt, W, bcol):
    return pl.pallas_call(
        _mm_body,
        grid=(_NB,),
        in_specs=[
            pl.BlockSpec((_D, _B), lambda i: (0, 0)),
            pl.BlockSpec((_V_BLK, _D), lambda i: (i, 0)),
            pl.BlockSpec((_V_BLK, 1), lambda i: (i, 0)),
        ],
        out_specs=pl.BlockSpec((_V_BLK, _B), lambda i: (i, 0)),
        out_shape=jax.ShapeDtypeStruct((_VOCAB, _B), jnp.float32),
        compiler_params=pltpu.CompilerParams(
            dimension_semantics=("arbitrary",),
        ),
    )(xt, W, bcol)


@jax.jit
def kernel(inputs, embeddings, W, b):
    idx = inputs.astype(jnp.int32)
    embeds = _make_sc_gather(_D, _B)(embeddings, idx)
    out_t = _tc_project_t(embeds.T, W, b.reshape(_VOCAB, 1))
    return out_t.T


# speedup vs baseline: 1.7952x; 1.0137x over previous
import functools

import jax
import jax.numpy as jnp
from jax import lax
from jax.experimental import pallas as pl
from jax.experimental.pallas import tpu as pltpu
from jax.experimental.pallas import tpu_sc as plsc

_VOCAB = 100000
_D = 64
_B = 1024
_V_BLK = 4096
_NB = pl.cdiv(_VOCAB, _V_BLK)


@functools.lru_cache(maxsize=None)
def _make_sc_gather(D, B):
    info = plsc.get_sparse_core_info()
    NC, NS = info.num_cores, info.num_subcores
    NW = NC * NS
    assert B % (8 * NW) == 0
    b_per_w = B // NW
    mesh = plsc.VectorSubcoreMesh(core_axis_name="c", subcore_axis_name="s")

    @functools.partial(
        pl.kernel,
        mesh=mesh,
        out_type=jax.ShapeDtypeStruct((B, D), jnp.float32),
        scratch_types=[
            pltpu.VMEM((b_per_w,), jnp.int32),
            pltpu.VMEM((b_per_w, D), jnp.float32),
            pltpu.SemaphoreType.DMA,
        ],
        compiler_params=pltpu.CompilerParams(use_tc_tiling_on_sc=False),
    )
    def gather(table_hbm, idx_hbm, out_hbm, idx_v, rows_v, sem):
        wid = lax.axis_index("s") * NC + lax.axis_index("c")
        base = wid * b_per_w
        pltpu.sync_copy(idx_hbm.at[pl.ds(base, b_per_w)], idx_v)
        pltpu.async_copy(table_hbm.at[idx_v], rows_v, sem).wait()
        pltpu.sync_copy(rows_v, out_hbm.at[pl.ds(base, b_per_w)])

    return gather


def _mm_body(xt_ref, w_ref, b_ref, o_ref):
    o_ref[...] = jnp.dot(
        w_ref[...], xt_ref[...], preferred_element_type=jnp.float32
    ) + b_ref[...]


def _tc_project_t(xt, W, bcol):
    return pl.pallas_call(
        _mm_body,
        grid=(_NB,),
        in_specs=[
            pl.BlockSpec((_D, _B), lambda i: (0, 0)),
            pl.BlockSpec((_V_BLK, _D), lambda i: (i, 0)),
            pl.BlockSpec((_V_BLK, 1), lambda i: (i, 0)),
        ],
        out_specs=pl.BlockSpec((_V_BLK, _B), lambda i: (i, 0)),
        out_shape=jax.ShapeDtypeStruct((_VOCAB, _B), jnp.float32),
        compiler_params=pltpu.CompilerParams(
            dimension_semantics=("arbitrary",),
        ),
    )(xt, W, bcol)


@jax.jit
def kernel(inputs, embeddings, W, b):
    idx = inputs.astype(jnp.int32)
    embeds = _make_sc_gather(_D, _B)(embeddings, idx)
    out_t = _tc_project_t(embeds.T, W, b.reshape(_VOCAB, 1))
    return out_t.T


# speedup vs baseline: 1.7986x; 1.0019x over previous
import functools

import jax
import jax.numpy as jnp
from jax import lax
from jax.experimental import pallas as pl
from jax.experimental.pallas import tpu as pltpu
from jax.experimental.pallas import tpu_sc as plsc

_VOCAB = 100000
_D = 64
_B = 1024
_V_BLK = 4096
_NB = pl.cdiv(_VOCAB, _V_BLK)


@functools.lru_cache(maxsize=None)
def _make_sc_gather(D, B):
    info = plsc.get_sparse_core_info()
    NC, NS = info.num_cores, info.num_subcores
    NW = NC * NS
    assert B % (8 * NW) == 0
    b_per_w = B // NW
    mesh = plsc.VectorSubcoreMesh(core_axis_name="c", subcore_axis_name="s")

    @functools.partial(
        pl.kernel,
        mesh=mesh,
        out_type=jax.ShapeDtypeStruct((B, D), jnp.float32),
        scratch_types=[
            pltpu.VMEM((b_per_w,), jnp.int32),
            pltpu.VMEM((b_per_w, D), jnp.float32),
            pltpu.SemaphoreType.DMA,
        ],
        compiler_params=pltpu.CompilerParams(use_tc_tiling_on_sc=False),
    )
    def gather(table_hbm, idx_hbm, out_hbm, idx_v, rows_v, sem):
        wid = lax.axis_index("s") * NC + lax.axis_index("c")
        base = wid * b_per_w
        pltpu.sync_copy(idx_hbm.at[pl.ds(base, b_per_w)], idx_v)
        pltpu.async_copy(table_hbm.at[idx_v], rows_v, sem).wait()
        pltpu.sync_copy(rows_v, out_hbm.at[pl.ds(base, b_per_w)])

    return gather


def _mm_body(xt_ref, w_ref, b_ref, o_ref):
    o_ref[...] = jnp.dot(
        w_ref[...], xt_ref[...], preferred_element_type=jnp.float32
    ) + b_ref[...]


def _tc_project_t(xt, W, bcol):
    return pl.pallas_call(
        _mm_body,
        grid=(_NB,),
        in_specs=[
            pl.BlockSpec((_D, _B), lambda i: (0, 0)),
            pl.BlockSpec((_V_BLK, _D), lambda i: (i, 0)),
            pl.BlockSpec((_V_BLK, 1), lambda i: (i, 0)),
        ],
        out_specs=pl.BlockSpec((_V_BLK, _B), lambda i: (i, 0)),
        out_shape=jax.ShapeDtypeStruct((_VOCAB, _B), jnp.float32),
        compiler_params=pltpu.CompilerParams(
            dimension_semantics=("parallel",),
        ),
    )(xt, W, bcol)


@jax.jit
def kernel(inputs, embeddings, W, b):
    idx = inputs.astype(jnp.int32)
    embeds = _make_sc_gather(_D, _B)(embeddings, idx)
    out_t = _tc_project_t(embeds.T, W, b.reshape(_VOCAB, 1))
    return out_t.T


# speedup vs baseline: 2.2044x; 1.2256x over previous
import functools

import jax
import jax.numpy as jnp
from jax import lax
from jax.experimental import pallas as pl
from jax.experimental.pallas import tpu as pltpu
from jax.experimental.pallas import tpu_sc as plsc

_VOCAB = 100000
_D = 64
_B = 1024
_V_BLK = 4096
_NB = pl.cdiv(_VOCAB, _V_BLK)


@functools.lru_cache(maxsize=None)
def _make_sc_gather(D, B):
    info = plsc.get_sparse_core_info()
    NC, NS = info.num_cores, info.num_subcores
    NW = NC * NS
    assert B % (8 * NW) == 0
    b_per_w = B // NW
    mesh = plsc.VectorSubcoreMesh(core_axis_name="c", subcore_axis_name="s")

    @functools.partial(
        pl.kernel,
        mesh=mesh,
        out_type=jax.ShapeDtypeStruct((B, D), jnp.float32),
        scratch_types=[
            pltpu.VMEM((b_per_w,), jnp.int32),
            pltpu.VMEM((b_per_w, D), jnp.float32),
            pltpu.SemaphoreType.DMA,
        ],
        compiler_params=pltpu.CompilerParams(use_tc_tiling_on_sc=False),
    )
    def gather(table_hbm, idx_hbm, out_hbm, idx_v, rows_v, sem):
        wid = lax.axis_index("s") * NC + lax.axis_index("c")
        base = wid * b_per_w
        pltpu.sync_copy(idx_hbm.at[pl.ds(base, b_per_w)], idx_v)
        pltpu.async_copy(table_hbm.at[idx_v], rows_v, sem).wait()
        pltpu.sync_copy(rows_v, out_hbm.at[pl.ds(base, b_per_w)])

    return gather


def _mm_body(xt_ref, w_ref, b_ref, o_ref):
    o_ref[...] = jnp.dot(
        w_ref[...], xt_ref[...], preferred_element_type=jnp.float32
    ) + lax.broadcast_in_dim(b_ref[...], (_V_BLK, 1), (0,))


def _tc_project_t(xt, W, bcol):
    return pl.pallas_call(
        _mm_body,
        grid=(_NB,),
        in_specs=[
            pl.BlockSpec((_D, _B), lambda i: (0, 0)),
            pl.BlockSpec((_V_BLK, _D), lambda i: (i, 0)),
            pl.BlockSpec((_V_BLK,), lambda i: (i,)),
        ],
        out_specs=pl.BlockSpec((_V_BLK, _B), lambda i: (i, 0)),
        out_shape=jax.ShapeDtypeStruct((_VOCAB, _B), jnp.float32),
        compiler_params=pltpu.CompilerParams(
            dimension_semantics=("parallel",),
        ),
    )(xt, W, bcol)


@jax.jit
def kernel(inputs, embeddings, W, b):
    idx = inputs.astype(jnp.int32)
    embeds = _make_sc_gather(_D, _B)(embeddings, idx)
    out_t = _tc_project_t(embeds.T, W, b)
    return out_t.T
